# Initial kernel scaffold; baseline (speedup 1.0000x reference)
import jax
import jax.numpy as jnp
from jax.experimental import pallas as pl


def kernel(nfeat, efeat, edge_index, elin1_W, elin1_b, elin2_W, elin2_b, conv1_b, conv2_b, out_W, out_b):
    raise NotImplementedError("write your pallas kernel here")


# speedup vs baseline: 2.3878x; 2.3878x over previous
import functools

import jax
import jax.numpy as jnp
from jax import lax
from jax.experimental import pallas as pl
from jax.experimental.pallas import tpu as pltpu
from jax.experimental.pallas import tpu_sc as plsc

H = 16
K = 16
TW = 384
SW = 2 * H
NW = 32
CHUNK = 128


def _mm_body(x_ref, w_ref, o0_ref, o1_ref, o2_ref):
    p = jnp.dot(x_ref[...], w_ref[...], preferred_element_type=jnp.float32)
    o0_ref[...] = p[:, 0:128]
    o1_ref[...] = p[:, 128:256]
    o2_ref[...] = p[:, 256:384]


def _node_table(x, w_ext, npad):
    n = x.shape[0]
    xp = jnp.pad(x, ((0, npad - n), (0, 0)))
    plane = jax.ShapeDtypeStruct((npad, 128), jnp.float32)
    return pl.pallas_call(
        _mm_body, out_shape=(plane, plane, plane))(xp, w_ext)


def _ext_weight(w_flat, b_flat, in_f):
    g = w_flat.reshape(in_f, H, K).transpose(0, 2, 1).reshape(in_f, K * H)
    b = b_flat.reshape(in_f, H)
    pad = jnp.zeros((in_f, TW - K * H - H), jnp.float32)
    return jnp.concatenate([g, b, pad], axis=1)


def _sc_conv(t0, t1, t2, efp, srcp, dstp, npad):
    epad = srcp.shape[0]
    per_w = epad // NW
    n_chunks = per_w // CHUNK
    rpt = npad // 16
    efrows = CHUNK * K // 128

    mesh = plsc.VectorSubcoreMesh(core_axis_name="c", subcore_axis_name="s",
                                  num_cores=2, num_subcores=16)

    @functools.partial(
        pl.kernel,
        out_type=jax.ShapeDtypeStruct((2, npad, SW), jnp.float32),
        mesh=mesh,
        compiler_params=pltpu.CompilerParams(use_tc_tiling_on_sc=False),
        scratch_types=[
            pltpu.VMEM((2, 3, CHUNK, 128), jnp.float32),
            pltpu.VMEM((2, efrows, 128), jnp.float32),
            pltpu.VMEM((2, CHUNK), jnp.int32),
            pltpu.VMEM((2, CHUNK), jnp.int32),
            pltpu.VMEM((CHUNK, SW), jnp.float32),
            pltpu.VMEM_SHARED((npad, SW), jnp.float32),
            pltpu.SemaphoreType.DMA((2,)),
            pltpu.SemaphoreType.DMA((2,)),
            pltpu.SemaphoreType.DMA((2,)),
        ],
    )
    def conv_kernel(t0_h, t1_h, t2_h, ef_h, src_h, dst_h, out_h,
                    rows_v, ef_v, src_v, dst_v, m_v, acc_s,
                    semg, sema, seme):
        cid = lax.axis_index("c")
        tid = lax.axis_index("s")
        wid = tid * 2 + cid
        wbase = wid * per_w
        efbase = wid * (per_w * K // 128)
        zv = jnp.zeros((16,), jnp.float32)

        def zrow(j, carry):
            m_v[j, pl.ds(0, 16)] = zv
            m_v[j, pl.ds(16, 16)] = zv
            return carry

        lax.fori_loop(0, CHUNK, zrow, 0)
        nfull, rem = divmod(rpt, CHUNK)
        for q in range(nfull):
            pltpu.sync_copy(m_v, acc_s.at[pl.ds(tid * rpt + q * CHUNK, CHUNK)])
        if rem:
            pltpu.sync_copy(m_v.at[pl.ds(0, rem)],
                            acc_s.at[pl.ds(tid * rpt + nfull * CHUNK, rem)])
        plsc.subcore_barrier()

        onehot = jnp.maximum(
            1.0 - lax.iota(jnp.int32, 16).astype(jnp.float32), 0.0)

        planes = (t0_h, t1_h, t2_h)

        def issue_src(i, slot):
            pltpu.async_copy(src_h.at[pl.ds(wbase + i * CHUNK, CHUNK)],
                             src_v.at[slot], sema.at[slot])

        def issue_efdst(i, slot):
            pltpu.async_copy(ef_h.at[pl.ds(efbase + i * efrows, efrows)],
                             ef_v.at[slot], seme.at[slot])
            pltpu.async_copy(dst_h.at[pl.ds(wbase + i * CHUNK, CHUNK)],
                             dst_v.at[slot], seme.at[slot])

        def issue_gather(slot):
            for r in range(3):
                pltpu.async_copy(planes[r].at[src_v.at[slot]],
                                 rows_v.at[slot, r], semg.at[slot])

        def wait_src(slot):
            pltpu.make_async_copy(src_h.at[pl.ds(0, CHUNK)],
                                  src_v.at[slot], sema.at[slot]).wait()

        def wait_efdst(slot):
            pltpu.make_async_copy(ef_h.at[pl.ds(0, efrows)],
                                  ef_v.at[slot], seme.at[slot]).wait()
            pltpu.make_async_copy(dst_h.at[pl.ds(0, CHUNK)],
                                  dst_v.at[slot], seme.at[slot]).wait()

        def wait_gather(slot):
            for r in range(3):
                pltpu.make_async_copy(planes[r].at[src_v.at[slot]],
                                      rows_v.at[slot, r], semg.at[slot]).wait()

        pltpu.sync_copy(src_h.at[pl.ds(wbase, CHUNK)], src_v.at[0])
        issue_gather(0)
        issue_efdst(0, 0)
        if n_chunks > 1:
            issue_src(1, 1)

        def chunk(i, carry):
            slot = lax.rem(i, 2)
            oslot = 1 - slot
            wait_gather(slot)

            @pl.when(i + 2 < n_chunks)
            def _():
                issue_src(i + 2, slot)

            @pl.when(i + 1 < n_chunks)
            def _():
                wait_src(oslot)
                issue_gather(oslot)
                issue_efdst(i + 1, oslot)

            wait_efdst(slot)

            @plsc.parallel_loop(0, CHUNK, unroll=2)
            def edge(j):
                efr = ef_v[slot, j // 8, pl.ds((j % 8) * K, K)]
                m0 = rows_v[slot, 2, j, pl.ds(0, H)]
                m1 = efr[1] * rows_v[slot, 0, j, pl.ds(1 * H, H)]
                m2 = efr[2] * rows_v[slot, 0, j, pl.ds(2 * H, H)]
                m3 = efr[3] * rows_v[slot, 0, j, pl.ds(3 * H, H)]
                m0 = m0 + efr[0] * rows_v[slot, 0, j, pl.ds(0, H)]
                for kk in range(4, K, 4):
                    m0 = m0 + efr[kk] * rows_v[slot, kk // 8, j,
                                               pl.ds((kk % 8) * H, H)]
                    m1 = m1 + efr[kk + 1] * rows_v[slot, (kk + 1) // 8, j,
                                                   pl.ds(((kk + 1) % 8) * H, H)]
                    m2 = m2 + efr[kk + 2] * rows_v[slot, (kk + 2) // 8, j,
                                                   pl.ds(((kk + 2) % 8) * H, H)]
                    m3 = m3 + efr[kk + 3] * rows_v[slot, (kk + 3) // 8, j,
                                                   pl.ds(((kk + 3) % 8) * H, H)]
                m_v[j, pl.ds(0, H)] = (m0 + m1) + (m2 + m3)
                m_v[j, pl.ds(H, 16)] = onehot

            pltpu.sync_copy(m_v, acc_s.at[dst_v.at[slot]], add=True)
            return carry

        lax.fori_loop(0, n_chunks, chunk, 0)
        plsc.subcore_barrier()
        pltpu.sync_copy(acc_s.at[pl.ds(tid * rpt, rpt)],
                        out_h.at[cid, pl.ds(tid * rpt, rpt)])

    return conv_kernel(t0, t1, t2, efp, srcp, dstp)


def _fin1_body(p_ref, g2_ref, b_ref, o0_ref, o1_ref, o2_ref):
    p = p_ref[0] + p_ref[1]
    s = p[:, :H]
    deg = p[:, H:H + 1]
    h = jnp.maximum(s / jnp.maximum(deg, 1.0) + b_ref[...], 0.0)
    p2 = jnp.dot(h, g2_ref[...], preferred_element_type=jnp.float32)
    o0_ref[...] = p2[:, 0:128]
    o1_ref[...] = p2[:, 128:256]
    o2_ref[...] = p2[:, 256:384]


def _fin2_body(p_ref, b_ref, w_ref, ob_ref, o_ref):
    p = p_ref[0] + p_ref[1]
    s = p[:, :H]
    deg = p[:, H:H + 1]
    h = jnp.maximum(s / jnp.maximum(deg, 1.0) + b_ref[...], 0.0)
    o_ref[...] = (jnp.dot(h, w_ref[...], preferred_element_type=jnp.float32)
                  + ob_ref[...])


def kernel(nfeat, efeat, edge_index, elin1_W, elin1_b, elin2_W, elin2_b,
           conv1_b, conv2_b, out_W, out_b):
    n = nfeat.shape[0]
    e = efeat.shape[0]
    in_f = nfeat.shape[1]

    npad = ((n + 1 + 127) // 128) * 128
    epad = ((e + NW * CHUNK - 1) // (NW * CHUNK)) * (NW * CHUNK)

    src = jnp.concatenate(
        [edge_index[0], jnp.zeros((epad - e,), jnp.int32)])
    dst = jnp.concatenate(
        [edge_index[1], jnp.full((epad - e,), n, jnp.int32)])
    efp = jnp.concatenate(
        [efeat, jnp.zeros((epad - e, K), jnp.float32)])
    efp = efp.reshape(epad * K // 128, 128)

    w1_ext = _ext_weight(elin1_W, elin1_b, in_f)
    t0, t1, t2 = _node_table(nfeat, w1_ext, npad)
    part1 = _sc_conv(t0, t1, t2, efp, src, dst, npad)

    w2_ext = _ext_weight(elin2_W, elin2_b, H)
    plane = jax.ShapeDtypeStruct((npad, 128), jnp.float32)
    u0, u1, u2 = pl.pallas_call(
        _fin1_body, out_shape=(plane, plane, plane))(part1, w2_ext, conv1_b)

    part2 = _sc_conv(u0, u1, u2, efp, src, dst, npad)

    out = pl.pallas_call(
        _fin2_body,
        out_shape=jax.ShapeDtypeStruct((npad, 1), jnp.float32),
    )(part2, conv2_b, out_W.T, out_b)
    return out[:n]


# speedup vs baseline: 3.3071x; 1.3850x over previous
import functools

import jax
import jax.numpy as jnp
from jax import lax
from jax.experimental import pallas as pl
from jax.experimental.pallas import tpu as pltpu
from jax.experimental.pallas import tpu_sc as plsc

H = 16
K = 16
WROW = K * H + H
SW = 2 * H
NW = 32
CHUNK = 128


def _mm_body(x_ref, w_ref, o_ref):
    o_ref[...] = jnp.dot(x_ref[...], w_ref[...],
                         preferred_element_type=jnp.float32)


def _node_table(x, w_ext, npad):
    n = x.shape[0]
    xp = jnp.pad(x, ((0, npad - n), (0, 0)))
    return pl.pallas_call(
        _mm_body,
        out_shape=jax.ShapeDtypeStruct((npad, WROW), jnp.float32),
    )(xp, w_ext)


def _ext_weight(w_flat, b_flat, in_f):
    g = w_flat.reshape(in_f, H, K).transpose(0, 2, 1).reshape(in_f, K * H)
    b = b_flat.reshape(in_f, H)
    return jnp.concatenate([g, b], axis=1)


def _sc_conv(table, efp, srcp, dstp, npad):
    epad = srcp.shape[0]
    per_w = epad // NW
    n_chunks = per_w // CHUNK
    rpt = npad // 16
    efrows = CHUNK * K // 128

    mesh = plsc.VectorSubcoreMesh(core_axis_name="c", subcore_axis_name="s",
                                  num_cores=2, num_subcores=16)

    @functools.partial(
        pl.kernel,
        out_type=jax.ShapeDtypeStruct((2, npad, SW), jnp.float32),
        mesh=mesh,
        compiler_params=pltpu.CompilerParams(use_tc_tiling_on_sc=False,
                                             skip_device_barrier=True),
        scratch_types=[
            pltpu.VMEM((2, CHUNK, WROW), jnp.float32),
            pltpu.VMEM((2, efrows, 128), jnp.float32),
            pltpu.VMEM((2, CHUNK), jnp.int32),
            pltpu.VMEM((3, CHUNK), jnp.int32),
            pltpu.VMEM((2, CHUNK, SW), jnp.float32),
            pltpu.VMEM((npad // 16, SW), jnp.float32),
            pltpu.VMEM_SHARED((npad, SW), jnp.float32),
            pltpu.SemaphoreType.DMA((2,)),
            pltpu.SemaphoreType.DMA((2,)),
            pltpu.SemaphoreType.DMA((2,)),
            pltpu.SemaphoreType.DMA((2,)),
        ],
    )
    def conv_kernel(table_h, ef_h, src_h, dst_h, out_h,
                    rows_v, ef_v, src_v, dst_v, m_v, z_v, acc_s,
                    semg, sema, seme, sems):
        cid = lax.axis_index("c")
        tid = lax.axis_index("s")
        wid = tid * 2 + cid
        wbase = wid * per_w
        efbase = wid * (per_w * K // 128)
        zv = jnp.zeros((16,), jnp.float32)

        def zrow(j, carry):
            z_v[j, pl.ds(0, 16)] = zv
            z_v[j, pl.ds(16, 16)] = zv
            return carry

        lax.fori_loop(0, rpt, zrow, 0)
        pltpu.sync_copy(z_v, acc_s.at[pl.ds(tid * rpt, rpt)])
        plsc.subcore_barrier()

        onehot = jnp.maximum(
            1.0 - lax.iota(jnp.int32, 16).astype(jnp.float32), 0.0)

        def issue_src(i, slot):
            pltpu.async_copy(src_h.at[pl.ds(wbase + i * CHUNK, CHUNK)],
                             src_v.at[slot], sema.at[slot])

        def issue_efdst(i, slot, dslot):
            pltpu.async_copy(ef_h.at[pl.ds(efbase + i * efrows, efrows)],
                             ef_v.at[slot], seme.at[slot])
            pltpu.async_copy(dst_h.at[pl.ds(wbase + i * CHUNK, CHUNK)],
                             dst_v.at[dslot], seme.at[slot])

        def issue_gather(slot):
            pltpu.async_copy(table_h.at[src_v.at[slot]], rows_v.at[slot],
                             semg.at[slot])

        def wait_src(slot):
            pltpu.make_async_copy(src_h.at[pl.ds(0, CHUNK)],
                                  src_v.at[slot], sema.at[slot]).wait()

        def wait_efdst(slot):
            pltpu.make_async_copy(ef_h.at[pl.ds(0, efrows)],
                                  ef_v.at[slot], seme.at[slot]).wait()
            pltpu.make_async_copy(dst_h.at[pl.ds(0, CHUNK)],
                                  dst_v.at[0], seme.at[slot]).wait()

        def issue_scatter(slot, dslot):
            pltpu.async_copy(m_v.at[slot], acc_s.at[dst_v.at[dslot]],
                             sems.at[slot], add=True)

        def wait_scatter(slot):
            pltpu.make_async_copy(m_v.at[slot], acc_s.at[dst_v.at[0]],
                                  sems.at[slot]).wait()

        def wait_gather(slot):
            pltpu.make_async_copy(table_h.at[src_v.at[slot]],
                                  rows_v.at[slot], semg.at[slot]).wait()

        pltpu.sync_copy(src_h.at[pl.ds(wbase, CHUNK)], src_v.at[0])
        issue_gather(0)
        issue_efdst(0, 0, 0)
        if n_chunks > 1:
            issue_src(1, 1)

        def chunk(i, carry):
            slot = lax.rem(i, 2)
            oslot = 1 - slot
            dslot = lax.rem(i, 3)
            ndslot = lax.rem(i + 1, 3)
            wait_gather(slot)

            @pl.when(i >= 2)
            def _():
                wait_scatter(slot)

            @pl.when(i + 2 < n_chunks)
            def _():
                issue_src(i + 2, slot)

            @pl.when(i + 1 < n_chunks)
            def _():
                wait_src(oslot)
                issue_gather(oslot)
                issue_efdst(i + 1, oslot, ndslot)

            wait_efdst(slot)

            @plsc.parallel_loop(0, CHUNK, unroll=2)
            def edge(j):
                efr = ef_v[slot, j // 8, pl.ds((j % 8) * K, K)]
                m0 = rows_v[slot, j, pl.ds(K * H, H)]
                m1 = efr[1] * rows_v[slot, j, pl.ds(1 * H, H)]
                m2 = efr[2] * rows_v[slot, j, pl.ds(2 * H, H)]
                m3 = efr[3] * rows_v[slot, j, pl.ds(3 * H, H)]
                m0 = m0 + efr[0] * rows_v[slot, j, pl.ds(0, H)]
                for kk in range(4, K, 4):
                    m0 = m0 + efr[kk] * rows_v[slot, j, pl.ds(kk * H, H)]
                    m1 = m1 + efr[kk + 1] * rows_v[slot, j,
                                                   pl.ds((kk + 1) * H, H)]
                    m2 = m2 + efr[kk + 2] * rows_v[slot, j,
                                                   pl.ds((kk + 2) * H, H)]
                    m3 = m3 + efr[kk + 3] * rows_v[slot, j,
                                                   pl.ds((kk + 3) * H, H)]
                m_v[slot, j, pl.ds(0, H)] = (m0 + m1) + (m2 + m3)
                m_v[slot, j, pl.ds(H, 16)] = onehot
            issue_scatter(slot, dslot)
            return carry

        lax.fori_loop(0, n_chunks, chunk, 0)
        wait_scatter(lax.rem(n_chunks - 1, 2))
        if n_chunks > 1:
            wait_scatter(lax.rem(n_chunks - 2, 2))
        plsc.subcore_barrier()
        pltpu.sync_copy(acc_s.at[pl.ds(tid * rpt, rpt)],
                        out_h.at[cid, pl.ds(tid * rpt, rpt)])

    return conv_kernel(table, efp, srcp, dstp)


def _fin1_body(p_ref, g2_ref, b_ref, o_ref):
    p = p_ref[0] + p_ref[1]
    s = p[:, :H]
    deg = p[:, H:H + 1]
    h = jnp.maximum(s / jnp.maximum(deg, 1.0) + b_ref[...], 0.0)
    o_ref[...] = jnp.dot(h, g2_ref[...], preferred_element_type=jnp.float32)


def _fin2_body(p_ref, b_ref, w_ref, ob_ref, o_ref):
    p = p_ref[0] + p_ref[1]
    s = p[:, :H]
    deg = p[:, H:H + 1]
    h = jnp.maximum(s / jnp.maximum(deg, 1.0) + b_ref[...], 0.0)
    o_ref[...] = (jnp.dot(h, w_ref[...], preferred_element_type=jnp.float32)
                  + ob_ref[...])


def kernel(nfeat, efeat, edge_index, elin1_W, elin1_b, elin2_W, elin2_b,
           conv1_b, conv2_b, out_W, out_b):
    n = nfeat.shape[0]
    e = efeat.shape[0]
    in_f = nfeat.shape[1]

    npad = ((n + 1 + 127) // 128) * 128
    epad = ((e + NW * CHUNK - 1) // (NW * CHUNK)) * (NW * CHUNK)

    src = jnp.concatenate(
        [edge_index[0], jnp.zeros((epad - e,), jnp.int32)])
    dst = jnp.concatenate(
        [edge_index[1], jnp.full((epad - e,), n, jnp.int32)])
    efp = jnp.concatenate(
        [efeat, jnp.zeros((epad - e, K), jnp.float32)])
    efp = efp.reshape(epad * K // 128, 128)

    w1_ext = _ext_weight(elin1_W, elin1_b, in_f)
    p1 = _node_table(nfeat, w1_ext, npad)
    part1 = _sc_conv(p1, efp, src, dst, npad)

    w2_ext = _ext_weight(elin2_W, elin2_b, H)
    p2 = pl.pallas_call(
        _fin1_body,
        out_shape=jax.ShapeDtypeStruct((npad, WROW), jnp.float32),
    )(part1, w2_ext, conv1_b)

    part2 = _sc_conv(p2, efp, src, dst, npad)

    out = pl.pallas_call(
        _fin2_body,
        out_shape=jax.ShapeDtypeStruct((npad, 1), jnp.float32),
    )(part2, conv2_b, out_W.T, out_b)
    return out[:n]


# speedup vs baseline: 3.6137x; 1.0927x over previous
import functools

import jax
import jax.numpy as jnp
from jax import lax
from jax.experimental import pallas as pl
from jax.experimental.pallas import tpu as pltpu
from jax.experimental.pallas import tpu_sc as plsc

H = 16
K = 16
WROW = K * H + H
SW = 2 * H
NW = 32
CHUNK = 128


def _mm_body(x_ref, w_ref, o_ref):
    o_ref[...] = jnp.dot(x_ref[...], w_ref[...],
                         preferred_element_type=jnp.float32)


def _node_table(x, w_ext, npad):
    n = x.shape[0]
    xp = jnp.pad(x, ((0, npad - n), (0, 0)))
    return pl.pallas_call(
        _mm_body,
        out_shape=jax.ShapeDtypeStruct((npad, WROW), jnp.float32),
    )(xp, w_ext)


def _ext_weight(w_flat, b_flat, in_f):
    g = w_flat.reshape(in_f, H, K).transpose(0, 2, 1).reshape(in_f, K * H)
    b = b_flat.reshape(in_f, H)
    return jnp.concatenate([g, b], axis=1)


def _sc_conv(table, efp, srcp, dstp, npad):
    epad = srcp.shape[0]
    pair_chunks = epad // CHUNK // 16
    c1_chunks = max(1, round(pair_chunks * 0.2725))
    c0_chunks = pair_chunks - c1_chunks
    n_chunks_max = c0_chunks
    rpt = npad // 16
    efrows = CHUNK * K // 128

    mesh = plsc.VectorSubcoreMesh(core_axis_name="c", subcore_axis_name="s",
                                  num_cores=2, num_subcores=16)

    @functools.partial(
        pl.kernel,
        out_type=jax.ShapeDtypeStruct((2, npad, SW), jnp.float32),
        mesh=mesh,
        compiler_params=pltpu.CompilerParams(use_tc_tiling_on_sc=False,
                                             skip_device_barrier=True),
        scratch_types=[
            pltpu.VMEM((2, CHUNK, WROW), jnp.float32),
            pltpu.VMEM((2, efrows, 128), jnp.float32),
            pltpu.VMEM((2, CHUNK), jnp.int32),
            pltpu.VMEM((3, CHUNK), jnp.int32),
            pltpu.VMEM((2, CHUNK, SW), jnp.float32),
            pltpu.VMEM((npad // 16, SW), jnp.float32),
            pltpu.VMEM_SHARED((npad, SW), jnp.float32),
            pltpu.SemaphoreType.DMA((2,)),
            pltpu.SemaphoreType.DMA((2,)),
            pltpu.SemaphoreType.DMA((2,)),
            pltpu.SemaphoreType.DMA((2,)),
        ],
    )
    def conv_kernel(table_h, ef_h, src_h, dst_h, out_h,
                    rows_v, ef_v, src_v, dst_v, m_v, z_v, acc_s,
                    semg, sema, seme, sems):
        cid = lax.axis_index("c")
        tid = lax.axis_index("s")
        nch = jnp.where(cid == 0, c0_chunks, c1_chunks)
        wchunk0 = jnp.where(cid == 0, tid * c0_chunks,
                            16 * c0_chunks + tid * c1_chunks)
        wbase = wchunk0 * CHUNK
        efbase = wchunk0 * efrows
        zv = jnp.zeros((16,), jnp.float32)

        def zrow(j, carry):
            z_v[j, pl.ds(0, 16)] = zv
            z_v[j, pl.ds(16, 16)] = zv
            return carry

        lax.fori_loop(0, rpt, zrow, 0)
        pltpu.sync_copy(z_v, acc_s.at[pl.ds(tid * rpt, rpt)])
        plsc.subcore_barrier()

        onehot = jnp.maximum(
            1.0 - lax.iota(jnp.int32, 16).astype(jnp.float32), 0.0)

        def issue_src(i, slot):
            pltpu.async_copy(src_h.at[pl.ds(wbase + i * CHUNK, CHUNK)],
                             src_v.at[slot], sema.at[slot])

        def issue_efdst(i, slot, dslot):
            pltpu.async_copy(ef_h.at[pl.ds(efbase + i * efrows, efrows)],
                             ef_v.at[slot], seme.at[slot])
            pltpu.async_copy(dst_h.at[pl.ds(wbase + i * CHUNK, CHUNK)],
                             dst_v.at[dslot], seme.at[slot])

        def issue_gather(slot):
            pltpu.async_copy(table_h.at[src_v.at[slot]], rows_v.at[slot],
                             semg.at[slot])

        def wait_src(slot):
            pltpu.make_async_copy(src_h.at[pl.ds(0, CHUNK)],
                                  src_v.at[slot], sema.at[slot]).wait()

        def wait_efdst(slot):
            pltpu.make_async_copy(ef_h.at[pl.ds(0, efrows)],
                                  ef_v.at[slot], seme.at[slot]).wait()
            pltpu.make_async_copy(dst_h.at[pl.ds(0, CHUNK)],
                                  dst_v.at[0], seme.at[slot]).wait()

        def issue_scatter(slot, dslot):
            pltpu.async_copy(m_v.at[slot], acc_s.at[dst_v.at[dslot]],
                             sems.at[slot], add=True)

        def wait_scatter(slot):
            pltpu.make_async_copy(m_v.at[slot], acc_s.at[dst_v.at[0]],
                                  sems.at[slot]).wait()

        def wait_gather(slot):
            pltpu.make_async_copy(table_h.at[src_v.at[slot]],
                                  rows_v.at[slot], semg.at[slot]).wait()

        pltpu.sync_copy(src_h.at[pl.ds(wbase, CHUNK)], src_v.at[0])
        issue_gather(0)
        issue_efdst(0, 0, 0)
        if c1_chunks > 1:
            issue_src(1, 1)

        def chunk(i, carry):
            slot = lax.rem(i, 2)
            oslot = 1 - slot
            dslot = lax.rem(i, 3)
            ndslot = lax.rem(i + 1, 3)
            wait_gather(slot)

            @pl.when(i >= 2)
            def _():
                wait_scatter(slot)

            @pl.when(i + 2 < nch)
            def _():
                issue_src(i + 2, slot)

            @pl.when(i + 1 < nch)
            def _():
                wait_src(oslot)
                issue_gather(oslot)
                issue_efdst(i + 1, oslot, ndslot)

            wait_efdst(slot)

            @plsc.parallel_loop(0, CHUNK, unroll=2)
            def edge(j):
                efr = ef_v[slot, j // 8, pl.ds((j % 8) * K, K)]
                m0 = rows_v[slot, j, pl.ds(K * H, H)]
                m1 = efr[1] * rows_v[slot, j, pl.ds(1 * H, H)]
                m2 = efr[2] * rows_v[slot, j, pl.ds(2 * H, H)]
                m3 = efr[3] * rows_v[slot, j, pl.ds(3 * H, H)]
                m0 = m0 + efr[0] * rows_v[slot, j, pl.ds(0, H)]
                for kk in range(4, K, 4):
                    m0 = m0 + efr[kk] * rows_v[slot, j, pl.ds(kk * H, H)]
                    m1 = m1 + efr[kk + 1] * rows_v[slot, j,
                                                   pl.ds((kk + 1) * H, H)]
                    m2 = m2 + efr[kk + 2] * rows_v[slot, j,
                                                   pl.ds((kk + 2) * H, H)]
                    m3 = m3 + efr[kk + 3] * rows_v[slot, j,
                                                   pl.ds((kk + 3) * H, H)]
                m_v[slot, j, pl.ds(0, H)] = (m0 + m1) + (m2 + m3)
                m_v[slot, j, pl.ds(H, 16)] = onehot
            issue_scatter(slot, dslot)
            return carry

        lax.fori_loop(0, nch, chunk, 0)
        wait_scatter(lax.rem(nch - 1, 2))
        if c1_chunks > 1:
            wait_scatter(lax.rem(nch - 2, 2))
        plsc.subcore_barrier()
        pltpu.sync_copy(acc_s.at[pl.ds(tid * rpt, rpt)],
                        out_h.at[cid, pl.ds(tid * rpt, rpt)])

    return conv_kernel(table, efp, srcp, dstp)


def _fin1_body(p_ref, g2_ref, b_ref, o_ref):
    p = p_ref[0] + p_ref[1]
    s = p[:, :H]
    deg = p[:, H:H + 1]
    h = jnp.maximum(s / jnp.maximum(deg, 1.0) + b_ref[...], 0.0)
    o_ref[...] = jnp.dot(h, g2_ref[...], preferred_element_type=jnp.float32)


def _fin2_body(p_ref, b_ref, w_ref, ob_ref, o_ref):
    p = p_ref[0] + p_ref[1]
    s = p[:, :H]
    deg = p[:, H:H + 1]
    h = jnp.maximum(s / jnp.maximum(deg, 1.0) + b_ref[...], 0.0)
    o_ref[...] = (jnp.dot(h, w_ref[...], preferred_element_type=jnp.float32)
                  + ob_ref[...])


def kernel(nfeat, efeat, edge_index, elin1_W, elin1_b, elin2_W, elin2_b,
           conv1_b, conv2_b, out_W, out_b):
    n = nfeat.shape[0]
    e = efeat.shape[0]
    in_f = nfeat.shape[1]

    npad = ((n + 1 + 127) // 128) * 128
    epad = ((e + NW * CHUNK - 1) // (NW * CHUNK)) * (NW * CHUNK)

    src = jnp.concatenate(
        [edge_index[0], jnp.zeros((epad - e,), jnp.int32)])
    dst = jnp.concatenate(
        [edge_index[1], jnp.full((epad - e,), n, jnp.int32)])
    efp = jnp.concatenate(
        [efeat, jnp.zeros((epad - e, K), jnp.float32)])
    efp = efp.reshape(epad * K // 128, 128)

    w1_ext = _ext_weight(elin1_W, elin1_b, in_f)
    p1 = _node_table(nfeat, w1_ext, npad)
    part1 = _sc_conv(p1, efp, src, dst, npad)

    w2_ext = _ext_weight(elin2_W, elin2_b, H)
    p2 = pl.pallas_call(
        _fin1_body,
        out_shape=jax.ShapeDtypeStruct((npad, WROW), jnp.float32),
    )(part1, w2_ext, conv1_b)

    part2 = _sc_conv(p2, efp, src, dst, npad)

    out = pl.pallas_call(
        _fin2_body,
        out_shape=jax.ShapeDtypeStruct((npad, 1), jnp.float32),
    )(part2, conv2_b, out_W.T, out_b)
    return out[:n]


# speedup vs baseline: 3.6505x; 1.0102x over previous
import functools

import jax
import jax.numpy as jnp
from jax import lax
from jax.experimental import pallas as pl
from jax.experimental.pallas import tpu as pltpu
from jax.experimental.pallas import tpu_sc as plsc

H = 16
K = 16
WROW = K * H + H
SW = 2 * H
NW = 32
CHUNK = 128


def _mm_body(x_ref, w_ref, o_ref):
    o_ref[...] = jnp.dot(x_ref[...], w_ref[...],
                         preferred_element_type=jnp.float32)


def _node_table(x, w_ext, npad):
    n = x.shape[0]
    xp = jnp.pad(x, ((0, npad - n), (0, 0)))
    return pl.pallas_call(
        _mm_body,
        out_shape=jax.ShapeDtypeStruct((npad, WROW), jnp.float32),
    )(xp, w_ext)


def _ext_weight(w_flat, b_flat, in_f):
    g = w_flat.reshape(in_f, H, K).transpose(0, 2, 1).reshape(in_f, K * H)
    b = b_flat.reshape(in_f, H)
    return jnp.concatenate([g, b], axis=1)


def _sc_conv(table, efp, srcp, dstp, npad):
    epad = srcp.shape[0]
    pair_chunks = epad // CHUNK // 16
    c1_chunks = max(1, round(pair_chunks * 0.05))
    c0_chunks = pair_chunks - c1_chunks
    n_chunks_max = c0_chunks
    rpt = npad // 16
    efrows = CHUNK * K // 128

    mesh = plsc.VectorSubcoreMesh(core_axis_name="c", subcore_axis_name="s",
                                  num_cores=2, num_subcores=16)

    @functools.partial(
        pl.kernel,
        out_type=jax.ShapeDtypeStruct((2, npad, SW), jnp.float32),
        mesh=mesh,
        compiler_params=pltpu.CompilerParams(use_tc_tiling_on_sc=False,
                                             skip_device_barrier=True),
        scratch_types=[
            pltpu.VMEM((2, CHUNK, WROW), jnp.float32),
            pltpu.VMEM((2, efrows, 128), jnp.float32),
            pltpu.VMEM((2, CHUNK), jnp.int32),
            pltpu.VMEM((3, CHUNK), jnp.int32),
            pltpu.VMEM((2, CHUNK, SW), jnp.float32),
            pltpu.VMEM((npad // 16, SW), jnp.float32),
            pltpu.VMEM_SHARED((npad, SW), jnp.float32),
            pltpu.SemaphoreType.DMA((2,)),
            pltpu.SemaphoreType.DMA((2,)),
            pltpu.SemaphoreType.DMA((2,)),
            pltpu.SemaphoreType.DMA((2,)),
        ],
    )
    def conv_kernel(table_h, ef_h, src_h, dst_h, out_h,
                    rows_v, ef_v, src_v, dst_v, m_v, z_v, acc_s,
                    semg, sema, seme, sems):
        cid = lax.axis_index("c")
        tid = lax.axis_index("s")
        nch = jnp.where(cid == 0, c0_chunks, c1_chunks)
        wchunk0 = jnp.where(cid == 0, tid * c0_chunks,
                            16 * c0_chunks + tid * c1_chunks)
        wbase = wchunk0 * CHUNK
        efbase = wchunk0 * efrows
        zv = jnp.zeros((16,), jnp.float32)

        def zrow(j, carry):
            z_v[j, pl.ds(0, 16)] = zv
            z_v[j, pl.ds(16, 16)] = zv
            return carry

        lax.fori_loop(0, rpt, zrow, 0)
        pltpu.sync_copy(z_v, acc_s.at[pl.ds(tid * rpt, rpt)])
        plsc.subcore_barrier()

        onehot = jnp.maximum(
            1.0 - lax.iota(jnp.int32, 16).astype(jnp.float32), 0.0)

        def issue_src(i, slot):
            pltpu.async_copy(src_h.at[pl.ds(wbase + i * CHUNK, CHUNK)],
                             src_v.at[slot], sema.at[slot])

        def issue_efdst(i, slot, dslot):
            pltpu.async_copy(ef_h.at[pl.ds(efbase + i * efrows, efrows)],
                             ef_v.at[slot], seme.at[slot])
            pltpu.async_copy(dst_h.at[pl.ds(wbase + i * CHUNK, CHUNK)],
                             dst_v.at[dslot], seme.at[slot])

        def issue_gather(slot):
            pltpu.async_copy(table_h.at[src_v.at[slot]], rows_v.at[slot],
                             semg.at[slot])

        def wait_src(slot):
            pltpu.make_async_copy(src_h.at[pl.ds(0, CHUNK)],
                                  src_v.at[slot], sema.at[slot]).wait()

        def wait_efdst(slot):
            pltpu.make_async_copy(ef_h.at[pl.ds(0, efrows)],
                                  ef_v.at[slot], seme.at[slot]).wait()
            pltpu.make_async_copy(dst_h.at[pl.ds(0, CHUNK)],
                                  dst_v.at[0], seme.at[slot]).wait()

        def issue_scatter(slot, dslot):
            pltpu.async_copy(m_v.at[slot], acc_s.at[dst_v.at[dslot]],
                             sems.at[slot], add=True)

        def wait_scatter(slot):
            pltpu.make_async_copy(m_v.at[slot], acc_s.at[dst_v.at[0]],
                                  sems.at[slot]).wait()

        def wait_gather(slot):
            pltpu.make_async_copy(table_h.at[src_v.at[slot]],
                                  rows_v.at[slot], semg.at[slot]).wait()

        pltpu.sync_copy(src_h.at[pl.ds(wbase, CHUNK)], src_v.at[0])
        issue_gather(0)
        issue_efdst(0, 0, 0)
        if c1_chunks > 1:
            issue_src(1, 1)

        def chunk(i, carry):
            slot = lax.rem(i, 2)
            oslot = 1 - slot
            dslot = lax.rem(i, 3)
            ndslot = lax.rem(i + 1, 3)
            wait_gather(slot)

            @pl.when(i >= 2)
            def _():
                wait_scatter(slot)

            @pl.when(i + 2 < nch)
            def _():
                issue_src(i + 2, slot)

            @pl.when(i + 1 < nch)
            def _():
                wait_src(oslot)
                issue_gather(oslot)
                issue_efdst(i + 1, oslot, ndslot)

            wait_efdst(slot)

            @plsc.parallel_loop(0, CHUNK, unroll=2)
            def edge(j):
                efr = ef_v[slot, j // 8, pl.ds((j % 8) * K, K)]
                m0 = rows_v[slot, j, pl.ds(K * H, H)]
                m1 = efr[1] * rows_v[slot, j, pl.ds(1 * H, H)]
                m2 = efr[2] * rows_v[slot, j, pl.ds(2 * H, H)]
                m3 = efr[3] * rows_v[slot, j, pl.ds(3 * H, H)]
                m0 = m0 + efr[0] * rows_v[slot, j, pl.ds(0, H)]
                for kk in range(4, K, 4):
                    m0 = m0 + efr[kk] * rows_v[slot, j, pl.ds(kk * H, H)]
                    m1 = m1 + efr[kk + 1] * rows_v[slot, j,
                                                   pl.ds((kk + 1) * H, H)]
                    m2 = m2 + efr[kk + 2] * rows_v[slot, j,
                                                   pl.ds((kk + 2) * H, H)]
                    m3 = m3 + efr[kk + 3] * rows_v[slot, j,
                                                   pl.ds((kk + 3) * H, H)]
                m_v[slot, j, pl.ds(0, H)] = (m0 + m1) + (m2 + m3)
                m_v[slot, j, pl.ds(H, 16)] = onehot
            issue_scatter(slot, dslot)
            return carry

        lax.fori_loop(0, nch, chunk, 0)
        wait_scatter(lax.rem(nch - 1, 2))
        if c1_chunks > 1:
            wait_scatter(lax.rem(nch - 2, 2))
        plsc.subcore_barrier()
        pltpu.sync_copy(acc_s.at[pl.ds(tid * rpt, rpt)],
                        out_h.at[cid, pl.ds(tid * rpt, rpt)])

    return conv_kernel(table, efp, srcp, dstp)


def _fin1_body(p_ref, g2_ref, b_ref, o_ref):
    p = p_ref[0] + p_ref[1]
    s = p[:, :H]
    deg = p[:, H:H + 1]
    h = jnp.maximum(s / jnp.maximum(deg, 1.0) + b_ref[...], 0.0)
    o_ref[...] = jnp.dot(h, g2_ref[...], preferred_element_type=jnp.float32)


def _fin2_body(p_ref, b_ref, w_ref, ob_ref, o_ref):
    p = p_ref[0] + p_ref[1]
    s = p[:, :H]
    deg = p[:, H:H + 1]
    h = jnp.maximum(s / jnp.maximum(deg, 1.0) + b_ref[...], 0.0)
    o_ref[...] = (jnp.dot(h, w_ref[...], preferred_element_type=jnp.float32)
                  + ob_ref[...])


def kernel(nfeat, efeat, edge_index, elin1_W, elin1_b, elin2_W, elin2_b,
           conv1_b, conv2_b, out_W, out_b):
    n = nfeat.shape[0]
    e = efeat.shape[0]
    in_f = nfeat.shape[1]

    npad = ((n + 1 + 127) // 128) * 128
    epad = ((e + NW * CHUNK - 1) // (NW * CHUNK)) * (NW * CHUNK)

    src = jnp.concatenate(
        [edge_index[0], jnp.zeros((epad - e,), jnp.int32)])
    dst = jnp.concatenate(
        [edge_index[1], jnp.full((epad - e,), n, jnp.int32)])
    efp = jnp.concatenate(
        [efeat, jnp.zeros((epad - e, K), jnp.float32)])
    efp = efp.reshape(epad * K // 128, 128)

    w1_ext = _ext_weight(elin1_W, elin1_b, in_f)
    p1 = _node_table(nfeat, w1_ext, npad)
    part1 = _sc_conv(p1, efp, src, dst, npad)

    w2_ext = _ext_weight(elin2_W, elin2_b, H)
    p2 = pl.pallas_call(
        _fin1_body,
        out_shape=jax.ShapeDtypeStruct((npad, WROW), jnp.float32),
    )(part1, w2_ext, conv1_b)

    part2 = _sc_conv(p2, efp, src, dst, npad)

    out = pl.pallas_call(
        _fin2_body,
        out_shape=jax.ShapeDtypeStruct((npad, 1), jnp.float32),
    )(part2, conv2_b, out_W.T, out_b)
    return out[:n]
